# Initial kernel scaffold; baseline (speedup 1.0000x reference)
import jax
import jax.numpy as jnp
from jax.experimental import pallas as pl


def kernel(base_feat, im_info, W_conv, b_conv, W_cls, b_cls, W_bbox, b_bbox, anchors):
    raise NotImplementedError("write your pallas kernel here")


# speedup vs baseline: 2.1730x; 2.1730x over previous
import jax
import jax.numpy as jnp
from jax.experimental import pallas as pl
from jax.experimental.pallas import tpu as pltpu

_B, _CIN, _T, _H, _W = 8, 512, 16, 15, 20
_A = 18
_N = _T * _H * _W * _A
_TOPN = 300
_NFRAMES = 16.0
_CH = 17280


def _conv3d(x, w, b, pad):
    y = jax.lax.conv_general_dilated(x, w, (1, 1, 1), pad,
                                     dimension_numbers=("NCDHW", "OIDHW", "NCDHW"))
    return y + b[None, :, None, None, None]


def _decode_body(a_ref, d_ref, hi_ref, o_ref):
    a = [a_ref[i:i + 1, :] for i in range(6)]
    d = [d_ref[0, i:i + 1, :] for i in range(6)]
    w = a[3] - a[0] + 1.0
    h = a[4] - a[1] + 1.0
    l = a[5] - a[2] + 1.0
    cx = a[0] + 0.5 * w
    cy = a[1] + 0.5 * h
    ct = a[2] + 0.5 * l
    pcx = d[0] * w + cx
    pcy = d[1] * h + cy
    pct = d[2] * l + ct
    pw = jnp.exp(d[3]) * w
    ph = jnp.exp(d[4]) * h
    plen = jnp.exp(d[5]) * l
    outs = [pcx - 0.5 * pw, pcy - 0.5 * ph, pct - 0.5 * plen,
            pcx + 0.5 * pw, pcy + 0.5 * ph, pct + 0.5 * plen]
    outs = [jnp.clip(outs[i], 0.0, hi_ref[0, 0, i]) for i in range(6)]
    o_ref[0] = jnp.concatenate(outs, axis=0)


def _decode_proposals(anchors_t, deltas_t, hi):
    grid = (_B, _N // _CH)
    return pl.pallas_call(
        _decode_body,
        grid=grid,
        in_specs=[
            pl.BlockSpec((6, _CH), lambda b, j: (0, j)),
            pl.BlockSpec((1, 6, _CH), lambda b, j: (b, 0, j)),
            pl.BlockSpec((1, 1, 6), lambda b, j: (b, 0, 0), memory_space=pltpu.SMEM),
        ],
        out_specs=pl.BlockSpec((1, 6, _CH), lambda b, j: (b, 0, j)),
        out_shape=jax.ShapeDtypeStruct((_B, 6, _N), jnp.float32),
        compiler_params=pltpu.CompilerParams(
            dimension_semantics=("parallel", "arbitrary")),
    )(anchors_t, deltas_t, hi)


def kernel(base_feat, im_info, W_conv, b_conv, W_cls, b_cls, W_bbox, b_bbox, anchors):
    rpn_conv1 = jax.nn.relu(_conv3d(base_feat, W_conv, b_conv, "SAME"))
    cls_score = _conv3d(rpn_conv1, W_cls, b_cls, "VALID")
    bbox_pred = _conv3d(rpn_conv1, W_bbox, b_bbox, "VALID")
    b, c, t, h, w = cls_score.shape
    score_rs = cls_score.reshape(b, 2, (c // 2) * t, h, w)
    prob = jax.nn.softmax(score_rs, axis=1).reshape(b, c, t, h, w)
    fg = prob[:, c // 2:, :, :, :]
    scores = jnp.transpose(fg, (0, 2, 3, 4, 1)).reshape(b, -1)

    thw = t * h * w
    deltas_t = (bbox_pred.reshape(b, _A, 6, thw)
                .transpose(0, 2, 3, 1).reshape(b, 6, _N))
    anchors_t = anchors.T
    lim = jnp.stack([im_info[:, 1] - 1.0, im_info[:, 0] - 1.0,
                     jnp.full((b,), _NFRAMES - 1.0, jnp.float32)], axis=-1)
    hi = jnp.concatenate([lim, lim], axis=-1).reshape(b, 1, 6)

    proposals_t = _decode_proposals(anchors_t, deltas_t, hi)

    _, topi = jax.lax.top_k(scores, _TOPN)
    top_props = jnp.take_along_axis(proposals_t, topi[:, None, :], axis=2)
    top_props = jnp.transpose(top_props, (0, 2, 1))
    batch_idx = jnp.broadcast_to(
        jnp.arange(b, dtype=jnp.float32)[:, None, None], (b, _TOPN, 1))
    rois = jnp.concatenate([batch_idx, top_props], axis=-1)
    return rois, prob, bbox_pred


# speedup vs baseline: 2.7031x; 1.2439x over previous
import jax
import jax.numpy as jnp
from jax.experimental import pallas as pl
from jax.experimental.pallas import tpu as pltpu

_B, _CIN, _T, _H, _W = 8, 512, 16, 15, 20
_A = 18
_P = _T * _H * _W
_N = _P * _A
_TOPN = 300
_NFRAMES = 16.0


def _conv3d(x, w, b, pad):
    y = jax.lax.conv_general_dilated(x, w, (1, 1, 1), pad,
                                     dimension_numbers=("NCDHW", "OIDHW", "NCDHW"))
    return y + b[None, :, None, None, None]


def _bbox_body(x_ref, w_ref, bias_ref, s_ref, c_ref, hi_ref, d_ref, p_ref):
    xb = x_ref[0].astype(jnp.bfloat16)
    wb = w_ref[...]
    d = jnp.dot(wb, xb, preferred_element_type=jnp.float32)
    d_ref[0] = d + bias_ref[...]

    rows = jax.lax.broadcasted_iota(jnp.int32, d.shape, 0)
    is_ctr = (rows % 6) < 3
    d_m3 = jnp.concatenate([d[3:], d[:3]], axis=0)
    d_p3 = jnp.concatenate([d[-3:], d[:-3]], axis=0)
    dctr = jnp.where(is_ctr, d, d_p3)
    dsz = jnp.where(is_ctr, d_m3, d)
    s = s_ref[...]
    c = c_ref[...]
    half = 0.5 * jnp.exp(dsz) * s
    ctr = dctr * s + c
    prop = jnp.where(is_ctr, ctr - half, ctr + half)
    rows3 = rows % 3
    hi = jnp.where(rows3 == 0, hi_ref[0, 0, 0],
                   jnp.where(rows3 == 1, hi_ref[0, 0, 1], hi_ref[0, 0, 2]))
    p_ref[0] = jnp.clip(prop, 0.0, hi)


def _bbox_decode(rpn_flat, wb_mat, bias_col, s_rows, c_rows, hi):
    out_sds = jax.ShapeDtypeStruct((_B, 108, _P), jnp.float32)
    return pl.pallas_call(
        _bbox_body,
        grid=(_B,),
        in_specs=[
            pl.BlockSpec((1, 512, _P), lambda b: (b, 0, 0)),
            pl.BlockSpec((108, 512), lambda b: (0, 0)),
            pl.BlockSpec((108, 1), lambda b: (0, 0)),
            pl.BlockSpec((108, _P), lambda b: (0, 0)),
            pl.BlockSpec((108, _P), lambda b: (0, 0)),
            pl.BlockSpec((1, 1, 6), lambda b: (b, 0, 0), memory_space=pltpu.SMEM),
        ],
        out_specs=[
            pl.BlockSpec((1, 108, _P), lambda b: (b, 0, 0)),
            pl.BlockSpec((1, 108, _P), lambda b: (b, 0, 0)),
        ],
        out_shape=[out_sds, out_sds],
        compiler_params=pltpu.CompilerParams(
            dimension_semantics=("parallel",)),
    )(rpn_flat, wb_mat, bias_col, s_rows, c_rows, hi)


def kernel(base_feat, im_info, W_conv, b_conv, W_cls, b_cls, W_bbox, b_bbox, anchors):
    b = base_feat.shape[0]
    rpn_conv1 = jax.nn.relu(_conv3d(base_feat, W_conv, b_conv, "SAME"))

    cls_score = _conv3d(rpn_conv1, W_cls, b_cls, "VALID")
    s0 = cls_score[:, :_A]
    s1 = cls_score[:, _A:]
    m = jnp.maximum(s0, s1)
    e0 = jnp.exp(s0 - m)
    e1 = jnp.exp(s1 - m)
    tot = e0 + e1
    pb = e0 / tot
    pf = e1 / tot
    prob = jnp.concatenate([pb, pf], axis=1)
    scores = jnp.transpose(pf, (0, 2, 3, 4, 1)).reshape(b, -1)
    _, topi = jax.lax.top_k(scores, _TOPN)

    anc = anchors.reshape(_P, _A, 6)
    aw = anc[..., 3] - anc[..., 0] + 1.0
    ah = anc[..., 4] - anc[..., 1] + 1.0
    al = anc[..., 5] - anc[..., 2] + 1.0
    acx = anc[..., 0] + 0.5 * aw
    acy = anc[..., 1] + 0.5 * ah
    act = anc[..., 2] + 0.5 * al
    s_rows = (jnp.stack([aw, ah, al, aw, ah, al], axis=-1)
              .transpose(1, 2, 0).reshape(108, _P))
    c_rows = (jnp.stack([acx, acy, act, acx, acy, act], axis=-1)
              .transpose(1, 2, 0).reshape(108, _P))
    lim = jnp.stack([im_info[:, 1] - 1.0, im_info[:, 0] - 1.0,
                     jnp.full((b,), _NFRAMES - 1.0, jnp.float32),
                     jnp.zeros((b,), jnp.float32),
                     jnp.zeros((b,), jnp.float32),
                     jnp.zeros((b,), jnp.float32)], axis=-1).reshape(b, 1, 6)
    wb_mat = W_bbox[:, :, 0, 0, 0].astype(jnp.bfloat16)
    bias_col = b_bbox.reshape(108, 1)
    rpn_flat = rpn_conv1.reshape(b, 512, _P)

    deltas, props = _bbox_decode(rpn_flat, wb_mat, bias_col, s_rows, c_rows, lim)
    bbox_pred = deltas.reshape(b, 108, _T, _H, _W)

    pk = topi // _A
    ak = topi % _A
    cols = jnp.take_along_axis(props, pk[:, None, :], axis=2)
    rsel = ak[:, None, :] * 6 + jnp.arange(6, dtype=topi.dtype)[None, :, None]
    top_props = jnp.take_along_axis(cols, rsel, axis=1)
    top_props = jnp.transpose(top_props, (0, 2, 1))
    batch_idx = jnp.broadcast_to(
        jnp.arange(b, dtype=jnp.float32)[:, None, None], (b, _TOPN, 1))
    rois = jnp.concatenate([batch_idx, top_props], axis=-1)
    return rois, prob, bbox_pred


# speedup vs baseline: 3.4210x; 1.2656x over previous
import jax
import jax.numpy as jnp
from jax.experimental import pallas as pl
from jax.experimental.pallas import tpu as pltpu

_B, _CIN, _T, _H, _W = 8, 512, 16, 15, 20
_A = 18
_P = _T * _H * _W
_N = _P * _A
_TOPN = 300
_CAND = 512
_NFRAMES = 16.0


def _conv3d(x, w, b, pad):
    y = jax.lax.conv_general_dilated(x, w, (1, 1, 1), pad,
                                     dimension_numbers=("NCDHW", "OIDHW", "NCDHW"))
    return y + b[None, :, None, None, None]


def _score_body(c_ref, prob_ref, tau_ref):
    s = c_ref[0]
    s0 = s[:_A]
    s1 = s[_A:]
    m = jnp.maximum(s0, s1)
    e0 = jnp.exp(s0 - m)
    e1 = jnp.exp(s1 - m)
    tot = e0 + e1
    pb = e0 / tot
    pf = e1 / tot
    prob_ref[0] = jnp.concatenate([pb, pf], axis=0)

    u = pltpu.bitcast(pf, jnp.int32)

    def body(_, lohi):
        lo, hi = lohi
        mid = (lo + hi) // 2
        cnt = jnp.sum((u >= mid).astype(jnp.float32))
        good = cnt >= float(_TOPN)
        return (jnp.where(good, mid, lo), jnp.where(good, hi, mid))

    lo, _hi = jax.lax.fori_loop(0, 31, body, (jnp.int32(0), jnp.int32(0x40000000)))
    tau_ref[0, 0, 0] = lo


def _scores_and_tau(cls_flat):
    return pl.pallas_call(
        _score_body,
        grid=(_B,),
        in_specs=[pl.BlockSpec((1, 36, _P), lambda b: (b, 0, 0))],
        out_specs=[
            pl.BlockSpec((1, 36, _P), lambda b: (b, 0, 0)),
            pl.BlockSpec((1, 1, 1), lambda b: (b, 0, 0), memory_space=pltpu.SMEM),
        ],
        out_shape=[
            jax.ShapeDtypeStruct((_B, 36, _P), jnp.float32),
            jax.ShapeDtypeStruct((_B, 1, 1), jnp.int32),
        ],
        compiler_params=pltpu.CompilerParams(dimension_semantics=("parallel",)),
    )(cls_flat)


def _bbox_body(x_ref, w_ref, bias_ref, s_ref, c_ref, hi_ref, d_ref, p_ref):
    xb = x_ref[0].astype(jnp.bfloat16)
    d = jnp.dot(w_ref[...], xb, preferred_element_type=jnp.float32)
    d = d + bias_ref[...]
    d_ref[0] = d

    rows = jax.lax.broadcasted_iota(jnp.int32, d.shape, 0)
    is_ctr = (rows % 6) < 3
    d_m3 = jnp.concatenate([d[3:], d[:3]], axis=0)
    d_p3 = jnp.concatenate([d[-3:], d[:-3]], axis=0)
    dctr = jnp.where(is_ctr, d, d_p3)
    dsz = jnp.where(is_ctr, d_m3, d)
    half = 0.5 * jnp.exp(dsz) * s_ref[...]
    ctr = dctr * s_ref[...] + c_ref[...]
    prop = jnp.where(is_ctr, ctr - half, ctr + half)
    rows3 = rows % 3
    hi = jnp.where(rows3 == 0, hi_ref[0, 0, 0],
                   jnp.where(rows3 == 1, hi_ref[0, 0, 1], hi_ref[0, 0, 2]))
    p_ref[0] = jnp.clip(prop, 0.0, hi)


def _bbox_decode(rpn_flat, wb_mat, bias_col, s_rows, c_rows, hi):
    out_sds = jax.ShapeDtypeStruct((_B, 108, _P), jnp.float32)
    return pl.pallas_call(
        _bbox_body,
        grid=(_B,),
        in_specs=[
            pl.BlockSpec((1, 512, _P), lambda b: (b, 0, 0)),
            pl.BlockSpec((108, 512), lambda b: (0, 0)),
            pl.BlockSpec((108, 1), lambda b: (0, 0)),
            pl.BlockSpec((108, _P), lambda b: (0, 0)),
            pl.BlockSpec((108, _P), lambda b: (0, 0)),
            pl.BlockSpec((1, 1, 6), lambda b: (b, 0, 0), memory_space=pltpu.SMEM),
        ],
        out_specs=[
            pl.BlockSpec((1, 108, _P), lambda b: (b, 0, 0)),
            pl.BlockSpec((1, 108, _P), lambda b: (b, 0, 0)),
        ],
        out_shape=[out_sds, out_sds],
        compiler_params=pltpu.CompilerParams(dimension_semantics=("parallel",)),
    )(rpn_flat, wb_mat, bias_col, s_rows, c_rows, hi)


def kernel(base_feat, im_info, W_conv, b_conv, W_cls, b_cls, W_bbox, b_bbox, anchors):
    b = base_feat.shape[0]
    rpn_conv1 = jax.nn.relu(_conv3d(base_feat, W_conv, b_conv, "SAME"))
    cls_score = _conv3d(rpn_conv1, W_cls, b_cls, "VALID")

    prob_flat, tau_i = _scores_and_tau(cls_score.reshape(b, 36, _P))
    prob = prob_flat.reshape(b, 36, _T, _H, _W)
    scores = jnp.transpose(prob_flat[:, _A:], (0, 2, 1)).reshape(b, _N)

    tau_f = jax.lax.bitcast_convert_type(tau_i[:, 0, 0], jnp.float32)
    mask = scores >= tau_f[:, None]
    cidx = jax.vmap(lambda mk: jnp.nonzero(mk, size=_CAND, fill_value=_N - 1)[0])(mask)
    cvals = jnp.take_along_axis(scores, cidx, axis=1)
    cnt = mask.sum(axis=1)
    cvals = jnp.where(jnp.arange(_CAND)[None, :] < cnt[:, None], cvals, -1.0)
    _, tkp = jax.lax.top_k(cvals, _TOPN)
    topi = jnp.take_along_axis(cidx, tkp, axis=1)

    anc = anchors.reshape(_P, _A, 6)
    aw = anc[..., 3] - anc[..., 0] + 1.0
    ah = anc[..., 4] - anc[..., 1] + 1.0
    al = anc[..., 5] - anc[..., 2] + 1.0
    acx = anc[..., 0] + 0.5 * aw
    acy = anc[..., 1] + 0.5 * ah
    act = anc[..., 2] + 0.5 * al
    s_rows = (jnp.stack([aw, ah, al, aw, ah, al], axis=-1)
              .transpose(1, 2, 0).reshape(108, _P))
    c_rows = (jnp.stack([acx, acy, act, acx, acy, act], axis=-1)
              .transpose(1, 2, 0).reshape(108, _P))
    lim = jnp.stack([im_info[:, 1] - 1.0, im_info[:, 0] - 1.0,
                     jnp.full((b,), _NFRAMES - 1.0, jnp.float32),
                     jnp.zeros((b,), jnp.float32),
                     jnp.zeros((b,), jnp.float32),
                     jnp.zeros((b,), jnp.float32)], axis=-1).reshape(b, 1, 6)
    wb_mat = W_bbox[:, :, 0, 0, 0].astype(jnp.bfloat16)
    bias_col = b_bbox.reshape(108, 1)
    rpn_flat = rpn_conv1.reshape(b, 512, _P)

    deltas, props = _bbox_decode(rpn_flat, wb_mat, bias_col, s_rows, c_rows, lim)
    bbox_pred = deltas.reshape(b, 108, _T, _H, _W)

    pk = topi // _A
    ak = topi % _A
    cols = jnp.take_along_axis(props, pk[:, None, :], axis=2)
    rsel = ak[:, None, :] * 6 + jnp.arange(6, dtype=topi.dtype)[None, :, None]
    top_props = jnp.take_along_axis(cols, rsel, axis=1)
    top_props = jnp.transpose(top_props, (0, 2, 1))
    batch_idx = jnp.broadcast_to(
        jnp.arange(b, dtype=jnp.float32)[:, None, None], (b, _TOPN, 1))
    rois = jnp.concatenate([batch_idx, top_props], axis=-1)
    return rois, prob, bbox_pred
